# baseline (speedup 1.0000x reference)
import jax
import jax.numpy as jnp
from jax.experimental import pallas as pl
from jax.experimental.pallas import tpu as pltpu

_N = 1_000_000
_D = 64
_BLOCK_ROWS = 8_000


def _rowdot_body(a_ref, b_ref, o_ref):
    a = a_ref[0]
    b = b_ref[0]
    o_ref[0, 0, :] = jnp.sum(a * b, axis=-1)


def kernel(gut, git):
    n_blocks = _N // _BLOCK_ROWS
    a3 = gut.reshape(n_blocks, _BLOCK_ROWS, _D)
    b3 = git.reshape(n_blocks, _BLOCK_ROWS, _D)
    out = pl.pallas_call(
        _rowdot_body,
        grid=(n_blocks,),
        in_specs=[
            pl.BlockSpec((1, _BLOCK_ROWS, _D), lambda i: (i, 0, 0)),
            pl.BlockSpec((1, _BLOCK_ROWS, _D), lambda i: (i, 0, 0)),
        ],
        out_specs=pl.BlockSpec((1, 1, _BLOCK_ROWS), lambda i: (i, 0, 0)),
        out_shape=jax.ShapeDtypeStruct((n_blocks, 1, _BLOCK_ROWS), jnp.float32),
        compiler_params=pltpu.CompilerParams(
            dimension_semantics=("arbitrary",),
        ),
    )(a3, b3)
    return out.reshape(_N)
